# baseline (speedup 1.0000x reference)
import functools

import jax
import jax.numpy as jnp
from jax import lax
from jax.experimental import pallas as pl
from jax.experimental.pallas import tpu as pltpu

_BB = 4096
_SUB = 1024


def _body(lab_ref, feat_hbm, cen_ref, out_ref, cent_bf, fbuf, sems, *, scale, C):
    nsub = _BB // _SUB
    cps = [
        pltpu.make_async_copy(
            feat_hbm.at[pl.ds(s * _SUB, _SUB), :], fbuf.at[s], sems.at[s])
        for s in range(nsub)
    ]
    for cp in cps:
        cp.start()
    cent_bf[...] = cen_ref[...].T.astype(jnp.bfloat16)
    ct = cent_bf[...]
    acc = jnp.zeros((8, 128), jnp.float32)
    for s in range(nsub):
        lab = lab_ref[0, 0, pl.ds(s * _SUB, _SUB)].astype(jnp.int16)
        onehot_t = jnp.where(
            lab[None, :] == lax.broadcasted_iota(jnp.int16, (C, _SUB), 0),
            jnp.bfloat16(1.0), jnp.bfloat16(0.0))
        g_t = jnp.dot(ct, onehot_t, preferred_element_type=jnp.float32)
        cps[s].wait()
        d = fbuf[s].T - g_t
        acc = acc + jnp.sum(
            (d * d).reshape(8, 16, _SUB).sum(axis=1).reshape(8, _SUB // 128, 128),
            axis=1)
    out_ref[0, 0] = jnp.sum(acc) * scale


def kernel(feat, labels, centers):
    B, D = feat.shape
    C = centers.shape[0]
    labels = labels.astype(jnp.int32).reshape(B // _BB, 1, _BB)
    out = pl.pallas_call(
        functools.partial(_body, scale=1.0 / (2.0 * B), C=C),
        grid=(B // _BB,),
        in_specs=[
            pl.BlockSpec((1, 1, _BB), lambda i: (i, 0, 0)),
            pl.BlockSpec(memory_space=pl.ANY),
            pl.BlockSpec((C, D), lambda i: (0, 0)),
        ],
        out_specs=pl.BlockSpec((1, 1), lambda i: (0, 0), memory_space=pltpu.SMEM),
        out_shape=jax.ShapeDtypeStruct((1, 1), jnp.float32),
        scratch_shapes=[
            pltpu.VMEM((D, C), jnp.bfloat16),
            pltpu.VMEM((B // _SUB, _SUB, D), jnp.float32),
            pltpu.SemaphoreType.DMA((B // _SUB,)),
        ],
        compiler_params=pltpu.CompilerParams(
            dimension_semantics=("arbitrary",),
        ),
    )(labels, feat, centers)
    return out[0, 0]


# speedup vs baseline: 1.2395x; 1.2395x over previous
import functools

import jax
import jax.numpy as jnp
from jax import lax
from jax.experimental import pallas as pl
from jax.experimental.pallas import tpu as pltpu

_BB = 4096
_SUB = 1024


def _body(lab_ref, feat_ref, cen_ref, out_ref, cent_bf, *, scale, C):
    cent_bf[...] = cen_ref[...].T.astype(jnp.bfloat16)
    ct = cent_bf[...]
    acc = jnp.zeros((8, _SUB), jnp.float32)
    iota_t = lax.broadcasted_iota(jnp.int16, (C, _SUB), 0)
    for s in range(_BB // _SUB):
        lab = lab_ref[0, 0, pl.ds(s * _SUB, _SUB)].astype(jnp.int16)
        onehot_t = jnp.where(
            lab[None, :] == iota_t,
            jnp.bfloat16(1.0), jnp.bfloat16(0.0))
        g_t = jnp.dot(ct, onehot_t, preferred_element_type=jnp.float32)
        d = feat_ref[pl.ds(s * _SUB, _SUB), :].T - g_t
        acc = acc + (d * d).reshape(16, 8, _SUB).sum(axis=0)
    out_ref[0, 0] = jnp.sum(acc) * scale


def kernel(feat, labels, centers):
    B, D = feat.shape
    C = centers.shape[0]
    labels = labels.astype(jnp.int32).reshape(B // _BB, 1, _BB)
    out = pl.pallas_call(
        functools.partial(_body, scale=1.0 / (2.0 * B), C=C),
        grid=(B // _BB,),
        in_specs=[
            pl.BlockSpec((1, 1, _BB), lambda i: (i, 0, 0)),
            pl.BlockSpec((_BB, D), lambda i: (i, 0)),
            pl.BlockSpec((C, D), lambda i: (0, 0)),
        ],
        out_specs=pl.BlockSpec((1, 1), lambda i: (0, 0), memory_space=pltpu.SMEM),
        out_shape=jax.ShapeDtypeStruct((1, 1), jnp.float32),
        scratch_shapes=[pltpu.VMEM((D, C), jnp.bfloat16)],
        compiler_params=pltpu.CompilerParams(
            dimension_semantics=("arbitrary",),
        ),
    )(labels, feat, centers)
    return out[0, 0]
